# Initial kernel scaffold; baseline (speedup 1.0000x reference)
import functools

import jax
import jax.numpy as jnp
from jax import lax
from jax.experimental import pallas as pl
from jax.experimental.pallas import tpu as pltpu
from jax.experimental.pallas import tpu_sc as plsc

N_NODES = 100000
N_EDGES = 6400000
NUM_TYPES = 64
FACTOR = 0.125

CHUNK = 2048
N_CHUNKS = N_EDGES // CHUNK
NW = 32
ACC_PAD = 131072
TILE_SLICE = ACC_PAD // 16

_mesh = plsc.VectorSubcoreMesh(core_axis_name="c", subcore_axis_name="s")


@functools.partial(
    pl.kernel,
    mesh=_mesh,
    out_type=jax.ShapeDtypeStruct((2, ACC_PAD), jnp.float32),
    scratch_types=[
        pltpu.VMEM((N_NODES,), jnp.int32),
        pltpu.VMEM((NUM_TYPES * NUM_TYPES,), jnp.float32),
        pltpu.VMEM((16, 128), jnp.int32),
        pltpu.VMEM((16, 128), jnp.int32),
        pltpu.VMEM((16, 128), jnp.float32),
        pltpu.VMEM((16, 128), jnp.float32),
        pltpu.VMEM((2048,), jnp.float32),
        pltpu.VMEM_SHARED((ACC_PAD,), jnp.float32),
        pltpu.SemaphoreType.DMA,
    ],
)
def _edge_scatter(center_hbm, neighbor_hbm, energy_hbm, species_hbm,
                  scales_hbm, out_hbm,
                  species_v, scales_v, c_buf, n_buf, e_buf, vals, zbuf,
                  acc, sem):
    cid = lax.axis_index("c")
    sid = lax.axis_index("s")
    wid = sid * 2 + cid

    zero16 = jnp.zeros((16,), jnp.float32)

    def _zb(i, carry):
        zbuf[pl.ds(i * 16, 16)] = zero16
        return carry

    lax.fori_loop(0, 128, _zb, 0)
    for t in range(TILE_SLICE // 2048):
        pltpu.sync_copy(zbuf, acc.at[pl.ds(sid * TILE_SLICE + t * 2048, 2048)])

    pltpu.sync_copy(species_hbm, species_v)
    pltpu.sync_copy(scales_hbm, scales_v)
    plsc.subcore_barrier()

    n_my_chunks = (N_CHUNKS - 1 - wid) // NW + 1

    def _chunk(t, carry):
        chunk = wid + t * NW
        pltpu.sync_copy(center_hbm.at[chunk], c_buf)
        pltpu.sync_copy(neighbor_hbm.at[chunk], n_buf)
        pltpu.sync_copy(energy_hbm.at[chunk], e_buf)

        for j in range(16):
            def _row(k, rc, j=j):
                sl = pl.ds(k * 16, 16)
                cc = c_buf[j, sl]
                nn = n_buf[j, sl]
                ee = e_buf[j, sl]
                csp = plsc.load_gather(species_v, [cc])
                nsp = plsc.load_gather(species_v, [nn])
                s = plsc.load_gather(scales_v, [csp * NUM_TYPES + nsp])
                vals[j, sl] = ee * s
                return rc

            lax.fori_loop(0, 8, _row, 0)

        copies = [
            pltpu.async_copy(vals.at[j], acc.at[c_buf.at[j]], sem, add=True)
            for j in range(16)
        ]
        for cp in copies:
            cp.wait()
        return carry

    lax.fori_loop(0, n_my_chunks, _chunk, 0)

    plsc.subcore_barrier()
    pltpu.sync_copy(acc.at[pl.ds(sid * TILE_SLICE, TILE_SLICE)],
                    out_hbm.at[cid, pl.ds(sid * TILE_SLICE, TILE_SLICE)])


def _combine_body(p_ref, o_ref):
    o_ref[...] = (p_ref[0] + p_ref[1]) * FACTOR


_combine = pl.pallas_call(
    _combine_body,
    out_shape=jax.ShapeDtypeStruct((ACC_PAD // 128, 128), jnp.float32),
)


@jax.jit
def kernel(edge_energy, edge_index, atom_types, per_edge_scales):
    center = edge_index[0].reshape(N_CHUNKS, 16, 128)
    neighbor = edge_index[1].reshape(N_CHUNKS, 16, 128)
    energy = edge_energy.reshape(N_CHUNKS, 16, 128)
    species = atom_types.reshape(N_NODES)
    scales = per_edge_scales.reshape(NUM_TYPES * NUM_TYPES)
    partials = _edge_scatter(center, neighbor, energy, species, scales)
    summed = _combine(partials.reshape(2, ACC_PAD // 128, 128))
    return summed.reshape(ACC_PAD)[:N_NODES].reshape(N_NODES, 1)


# speedup vs baseline: 383.1119x; 383.1119x over previous
import functools

import jax
import jax.numpy as jnp
from jax import lax
from jax.experimental import pallas as pl
from jax.experimental.pallas import tpu as pltpu
from jax.experimental.pallas import tpu_sc as plsc

N_NODES = 100000
N_EDGES = 6400000
NUM_TYPES = 64
FACTOR = 0.125

CHUNK = 2048
N_CHUNKS = N_EDGES // CHUNK
NW = 32
ACC_PAD = 131072
TILE_SLICE = ACC_PAD // 16

_mesh = plsc.VectorSubcoreMesh(core_axis_name="c", subcore_axis_name="s")


@functools.partial(
    pl.kernel,
    mesh=_mesh,
    compiler_params=pltpu.CompilerParams(needs_layout_passes=False),
    out_type=jax.ShapeDtypeStruct((2, ACC_PAD), jnp.float32),
    scratch_types=[
        pltpu.VMEM((N_NODES,), jnp.int32),
        pltpu.VMEM((NUM_TYPES * NUM_TYPES,), jnp.float32),
        pltpu.VMEM((16, 128), jnp.int32),
        pltpu.VMEM((16, 128), jnp.int32),
        pltpu.VMEM((16, 128), jnp.float32),
        pltpu.VMEM((16, 128), jnp.float32),
        pltpu.VMEM((2048,), jnp.float32),
        pltpu.VMEM_SHARED((ACC_PAD,), jnp.float32),
        pltpu.SemaphoreType.DMA,
    ],
)
def _edge_scatter(center_hbm, neighbor_hbm, energy_hbm, species_hbm,
                  scales_hbm, out_hbm,
                  species_v, scales_v, c_buf, n_buf, e_buf, vals, zbuf,
                  acc, sem):
    cid = lax.axis_index("c")
    sid = lax.axis_index("s")
    wid = sid * 2 + cid

    zero16 = jnp.zeros((16,), jnp.float32)

    def _zb(i, carry):
        zbuf[pl.ds(i * 16, 16)] = zero16
        return carry

    lax.fori_loop(0, 128, _zb, 0)
    for t in range(TILE_SLICE // 2048):
        pltpu.sync_copy(zbuf, acc.at[pl.ds(sid * TILE_SLICE + t * 2048, 2048)])

    pltpu.sync_copy(species_hbm, species_v)
    pltpu.sync_copy(scales_hbm, scales_v)
    plsc.subcore_barrier()

    n_my_chunks = (N_CHUNKS - 1 - wid) // NW + 1

    def _chunk(t, carry):
        chunk = wid + t * NW
        pltpu.sync_copy(center_hbm.at[chunk], c_buf)
        pltpu.sync_copy(neighbor_hbm.at[chunk], n_buf)
        pltpu.sync_copy(energy_hbm.at[chunk], e_buf)

        for j in range(16):
            def _row(k, rc, j=j):
                sl = pl.ds(k * 16, 16)
                cc = c_buf[j, sl]
                nn = n_buf[j, sl]
                ee = e_buf[j, sl]
                csp = plsc.load_gather(species_v, [cc])
                nsp = plsc.load_gather(species_v, [nn])
                s = plsc.load_gather(scales_v, [csp * NUM_TYPES + nsp])
                vals[j, sl] = ee * s
                return rc

            lax.fori_loop(0, 8, _row, 0)

        copies = [
            pltpu.async_copy(vals.at[j], acc.at[c_buf.at[j]], sem, add=True)
            for j in range(16)
        ]
        for cp in copies:
            cp.wait()
        return carry

    lax.fori_loop(0, n_my_chunks, _chunk, 0)

    plsc.subcore_barrier()
    pltpu.sync_copy(acc.at[pl.ds(sid * TILE_SLICE, TILE_SLICE)],
                    out_hbm.at[cid, pl.ds(sid * TILE_SLICE, TILE_SLICE)])


def _combine_body(p_ref, o_ref):
    o_ref[...] = (p_ref[0] + p_ref[1]) * FACTOR


_combine = pl.pallas_call(
    _combine_body,
    out_shape=jax.ShapeDtypeStruct((ACC_PAD // 128, 128), jnp.float32),
)


@jax.jit
def kernel(edge_energy, edge_index, atom_types, per_edge_scales):
    center = edge_index[0].reshape(N_CHUNKS, 16, 128)
    neighbor = edge_index[1].reshape(N_CHUNKS, 16, 128)
    energy = edge_energy.reshape(N_CHUNKS, 16, 128)
    species = atom_types.reshape(N_NODES)
    scales = per_edge_scales.reshape(NUM_TYPES * NUM_TYPES)
    partials = _edge_scatter(center, neighbor, energy, species, scales)
    summed = _combine(partials.reshape(2, ACC_PAD // 128, 128))
    return summed.reshape(ACC_PAD)[:N_NODES].reshape(N_NODES, 1)


# speedup vs baseline: 594.6568x; 1.5522x over previous
import functools

import jax
import jax.numpy as jnp
from jax import lax
from jax.experimental import pallas as pl
from jax.experimental.pallas import tpu as pltpu
from jax.experimental.pallas import tpu_sc as plsc

N_NODES = 100000
N_EDGES = 6400000
NUM_TYPES = 64
FACTOR = 0.125

ROWS = 8
CHUNK = ROWS * 128
N_CHUNKS = N_EDGES // CHUNK
NW = 32
ACC_PAD = 100352
TILE_SLICE = ACC_PAD // 16

_mesh = plsc.VectorSubcoreMesh(core_axis_name="c", subcore_axis_name="s")


@functools.partial(
    pl.kernel,
    mesh=_mesh,
    compiler_params=pltpu.CompilerParams(needs_layout_passes=False),
    out_type=jax.ShapeDtypeStruct((2, ACC_PAD), jnp.float32),
    scratch_types=[
        pltpu.VMEM((N_NODES,), jnp.int32),
        pltpu.VMEM((NUM_TYPES * NUM_TYPES,), jnp.float32),
        pltpu.VMEM((3, ROWS, 128), jnp.int32),
        pltpu.VMEM((2, ROWS, 128), jnp.int32),
        pltpu.VMEM((2, ROWS, 128), jnp.float32),
        pltpu.VMEM((3, ROWS, 128), jnp.float32),
        pltpu.VMEM_SHARED((ACC_PAD,), jnp.float32),
        pltpu.SemaphoreType.DMA,
        pltpu.SemaphoreType.DMA,
    ],
)
def _edge_scatter(center_hbm, neighbor_hbm, energy_hbm, species_hbm,
                  scales_hbm, zeros_hbm, out_hbm,
                  species_v, scales_v, c_buf, n_buf, e_buf, vals,
                  acc, in_sem, sc_sem):
    cid = lax.axis_index("c")
    sid = lax.axis_index("s")
    wid = sid * 2 + cid

    pltpu.sync_copy(zeros_hbm, acc.at[pl.ds(sid * TILE_SLICE, TILE_SLICE)])

    pltpu.sync_copy(species_hbm, species_v)
    pltpu.sync_copy(scales_hbm, scales_v)
    plsc.subcore_barrier()

    n_my_chunks = (N_CHUNKS - 1 - wid) // NW + 1

    def _start_inputs(t, p3, q2):
        chunk = wid + t * NW
        pltpu.make_async_copy(center_hbm.at[chunk], c_buf.at[p3], in_sem).start()
        pltpu.make_async_copy(neighbor_hbm.at[chunk], n_buf.at[q2], in_sem).start()
        pltpu.make_async_copy(energy_hbm.at[chunk], e_buf.at[q2], in_sem).start()

    def _wait_inputs(t, p3, q2):
        chunk = wid + t * NW
        pltpu.make_async_copy(center_hbm.at[chunk], c_buf.at[p3], in_sem).wait()
        pltpu.make_async_copy(neighbor_hbm.at[chunk], n_buf.at[q2], in_sem).wait()
        pltpu.make_async_copy(energy_hbm.at[chunk], e_buf.at[q2], in_sem).wait()

    def _drain_scatters(p3):
        for j in range(ROWS):
            pltpu.make_async_copy(
                vals.at[p3, j], acc.at[c_buf.at[p3, j]], sc_sem).wait()

    _start_inputs(0, 0, 0)

    def _chunk(t, carry):
        p3 = lax.rem(t, 3)
        q2 = lax.rem(t, 2)
        @pl.when(t >= 2)
        def _():
            _drain_scatters(lax.rem(t + 1, 3))

        @pl.when(t + 1 < n_my_chunks)
        def _():
            _start_inputs(t + 1, lax.rem(t + 1, 3), 1 - q2)

        _wait_inputs(t, p3, q2)

        for j in range(ROWS):
            def _row(k, rc, j=j):
                sl = pl.ds(k * 16, 16)
                cc = c_buf[p3, j, sl]
                nn = n_buf[q2, j, sl]
                ee = e_buf[q2, j, sl]
                csp = plsc.load_gather(species_v, [cc])
                nsp = plsc.load_gather(species_v, [nn])
                s = plsc.load_gather(scales_v, [csp * NUM_TYPES + nsp])
                vals[p3, j, sl] = ee * s
                return rc

            lax.fori_loop(0, 8, _row, 0)

        for j in range(ROWS):
            pltpu.async_copy(vals.at[p3, j], acc.at[c_buf.at[p3, j]],
                             sc_sem, add=True)
        return carry

    lax.fori_loop(0, n_my_chunks, _chunk, 0)
    _drain_scatters(lax.rem(n_my_chunks - 2, 3))
    _drain_scatters(lax.rem(n_my_chunks - 1, 3))

    plsc.subcore_barrier()
    pltpu.sync_copy(acc.at[pl.ds(sid * TILE_SLICE, TILE_SLICE)],
                    out_hbm.at[cid, pl.ds(sid * TILE_SLICE, TILE_SLICE)])


def _combine_body(p_ref, o_ref):
    o_ref[...] = (p_ref[0] + p_ref[1]) * FACTOR


_combine = pl.pallas_call(
    _combine_body,
    out_shape=jax.ShapeDtypeStruct((ACC_PAD // 128, 128), jnp.float32),
)


@jax.jit
def kernel(edge_energy, edge_index, atom_types, per_edge_scales):
    center = edge_index[0].reshape(N_CHUNKS, ROWS, 128)
    neighbor = edge_index[1].reshape(N_CHUNKS, ROWS, 128)
    energy = edge_energy.reshape(N_CHUNKS, ROWS, 128)
    species = atom_types.reshape(N_NODES)
    scales = per_edge_scales.reshape(NUM_TYPES * NUM_TYPES)
    zeros = jnp.zeros((TILE_SLICE,), jnp.float32)
    partials = _edge_scatter(center, neighbor, energy, species, scales, zeros)
    summed = _combine(partials.reshape(2, ACC_PAD // 128, 128))
    return summed.reshape(ACC_PAD)[:N_NODES].reshape(N_NODES, 1)


# speedup vs baseline: 995.8360x; 1.6746x over previous
import functools

import jax
import jax.numpy as jnp
from jax import lax
from jax.experimental import pallas as pl
from jax.experimental.pallas import tpu as pltpu
from jax.experimental.pallas import tpu_sc as plsc

N_NODES = 100000
N_EDGES = 6400000
NUM_TYPES = 64
FACTOR = 0.125

ROWS = 8
CHUNK = ROWS * 128
N_CHUNKS = N_EDGES // CHUNK
NW = 32
ACC_PAD = 100352
TILE_SLICE = ACC_PAD // 16

_mesh = plsc.VectorSubcoreMesh(core_axis_name="c", subcore_axis_name="s")


@functools.partial(
    pl.kernel,
    mesh=_mesh,
    compiler_params=pltpu.CompilerParams(needs_layout_passes=False),
    out_type=jax.ShapeDtypeStruct((2, ACC_PAD), jnp.float32),
    scratch_types=[
        pltpu.VMEM((N_NODES,), jnp.int32),
        pltpu.VMEM((NUM_TYPES * NUM_TYPES,), jnp.float32),
        pltpu.VMEM((3, ROWS, 128), jnp.int32),
        pltpu.VMEM((2, ROWS, 128), jnp.int32),
        pltpu.VMEM((2, ROWS, 128), jnp.float32),
        pltpu.VMEM((3, ROWS, 128), jnp.float32),
        pltpu.VMEM_SHARED((ACC_PAD,), jnp.float32),
        pltpu.SemaphoreType.DMA,
        pltpu.SemaphoreType.DMA,
    ],
)
def _edge_scatter(center_hbm, neighbor_hbm, energy_hbm, species_hbm,
                  scales_hbm, zeros_hbm, out_hbm,
                  species_v, scales_v, c_buf, n_buf, e_buf, vals,
                  acc, in_sem, sc_sem):
    cid = lax.axis_index("c")
    sid = lax.axis_index("s")
    wid = sid * 2 + cid

    pltpu.sync_copy(zeros_hbm, acc.at[pl.ds(sid * TILE_SLICE, TILE_SLICE)])

    pltpu.sync_copy(species_hbm, species_v)
    pltpu.sync_copy(scales_hbm, scales_v)
    plsc.subcore_barrier()

    n_my_chunks = (N_CHUNKS - 1 - wid) // NW + 1

    def _start_inputs(t, p3, q2):
        chunk = wid + t * NW
        pltpu.make_async_copy(center_hbm.at[chunk], c_buf.at[p3], in_sem).start()
        pltpu.make_async_copy(neighbor_hbm.at[chunk], n_buf.at[q2], in_sem).start()
        pltpu.make_async_copy(energy_hbm.at[chunk], e_buf.at[q2], in_sem).start()

    def _wait_inputs(t, p3, q2):
        chunk = wid + t * NW
        pltpu.make_async_copy(center_hbm.at[chunk], c_buf.at[p3], in_sem).wait()
        pltpu.make_async_copy(neighbor_hbm.at[chunk], n_buf.at[q2], in_sem).wait()
        pltpu.make_async_copy(energy_hbm.at[chunk], e_buf.at[q2], in_sem).wait()

    def _drain_scatters(p3):
        for j in range(ROWS):
            pltpu.make_async_copy(
                vals.at[p3, j], acc.at[c_buf.at[p3, j]], sc_sem).wait()

    _start_inputs(0, 0, 0)

    def _chunk(t, carry):
        p3 = lax.rem(t, 3)
        q2 = lax.rem(t, 2)
        @pl.when(t >= 2)
        def _():
            _drain_scatters(lax.rem(t + 1, 3))

        @pl.when(t + 1 < n_my_chunks)
        def _():
            _start_inputs(t + 1, lax.rem(t + 1, 3), 1 - q2)

        _wait_inputs(t, p3, q2)

        @plsc.parallel_loop(0, CHUNK, 16, unroll=8)
        def _body(i):
            j = lax.shift_right_logical(i, 7)
            sl = pl.ds(lax.bitwise_and(i, 127), 16)
            cc = c_buf[p3, j, sl]
            nn = n_buf[q2, j, sl]
            ee = e_buf[q2, j, sl]
            csp = plsc.load_gather(species_v, [cc])
            nsp = plsc.load_gather(species_v, [nn])
            s = plsc.load_gather(scales_v, [csp * NUM_TYPES + nsp])
            vals[p3, j, sl] = ee * s

        for j in range(ROWS):
            pltpu.async_copy(vals.at[p3, j], acc.at[c_buf.at[p3, j]],
                             sc_sem, add=True)
        return carry

    lax.fori_loop(0, n_my_chunks, _chunk, 0)
    _drain_scatters(lax.rem(n_my_chunks - 2, 3))
    _drain_scatters(lax.rem(n_my_chunks - 1, 3))

    plsc.subcore_barrier()
    pltpu.sync_copy(acc.at[pl.ds(sid * TILE_SLICE, TILE_SLICE)],
                    out_hbm.at[cid, pl.ds(sid * TILE_SLICE, TILE_SLICE)])


def _combine_body(p_ref, o_ref):
    o_ref[...] = (p_ref[0] + p_ref[1]) * FACTOR


_combine = pl.pallas_call(
    _combine_body,
    out_shape=jax.ShapeDtypeStruct((ACC_PAD // 128, 128), jnp.float32),
)


@jax.jit
def kernel(edge_energy, edge_index, atom_types, per_edge_scales):
    center = edge_index[0].reshape(N_CHUNKS, ROWS, 128)
    neighbor = edge_index[1].reshape(N_CHUNKS, ROWS, 128)
    energy = edge_energy.reshape(N_CHUNKS, ROWS, 128)
    species = atom_types.reshape(N_NODES)
    scales = per_edge_scales.reshape(NUM_TYPES * NUM_TYPES)
    zeros = jnp.zeros((TILE_SLICE,), jnp.float32)
    partials = _edge_scatter(center, neighbor, energy, species, scales, zeros)
    summed = _combine(partials.reshape(2, ACC_PAD // 128, 128))
    return summed.reshape(ACC_PAD)[:N_NODES].reshape(N_NODES, 1)


# speedup vs baseline: 1099.8773x; 1.1045x over previous
import functools

import jax
import jax.numpy as jnp
from jax import lax
from jax.experimental import pallas as pl
from jax.experimental.pallas import tpu as pltpu
from jax.experimental.pallas import tpu_sc as plsc

N_NODES = 100000
N_EDGES = 6400000
NUM_TYPES = 64
FACTOR = 0.125

ROWS = 8
CHUNK = ROWS * 128
N_CHUNKS = N_EDGES // CHUNK
NW = 32
ACC_PAD = 100352
TILE_SLICE = ACC_PAD // 16

_mesh = plsc.VectorSubcoreMesh(core_axis_name="c", subcore_axis_name="s")


@functools.partial(
    pl.kernel,
    mesh=_mesh,
    compiler_params=pltpu.CompilerParams(needs_layout_passes=False),
    out_type=jax.ShapeDtypeStruct((2, ACC_PAD), jnp.float32),
    scratch_types=[
        pltpu.VMEM((N_NODES,), jnp.int32),
        pltpu.VMEM((NUM_TYPES * NUM_TYPES,), jnp.float32),
        pltpu.VMEM((3, ROWS, 128), jnp.int32),
        pltpu.VMEM((2, ROWS, 128), jnp.int32),
        pltpu.VMEM((2, ROWS, 128), jnp.float32),
        pltpu.VMEM((3, ROWS, 128), jnp.float32),
        pltpu.VMEM_SHARED((ACC_PAD,), jnp.float32),
        pltpu.SemaphoreType.DMA,
        pltpu.SemaphoreType.DMA,
    ],
)
def _edge_scatter(edges_hbm, energy_hbm, species_hbm,
                  scales_hbm, zeros_hbm, out_hbm,
                  species_v, scales_v, c_buf, n_buf, e_buf, vals,
                  acc, in_sem, sc_sem):
    cid = lax.axis_index("c")
    sid = lax.axis_index("s")
    wid = sid * 2 + cid

    pltpu.sync_copy(zeros_hbm, acc.at[pl.ds(sid * TILE_SLICE, TILE_SLICE)])

    pltpu.sync_copy(species_hbm, species_v)
    pltpu.sync_copy(scales_hbm, scales_v)
    plsc.subcore_barrier()

    n_my_chunks = (N_CHUNKS - 1 - wid) // NW + 1

    def _start_inputs(t, p3, q2):
        chunk = wid + t * NW
        pltpu.make_async_copy(edges_hbm.at[0, chunk], c_buf.at[p3], in_sem).start()
        pltpu.make_async_copy(edges_hbm.at[1, chunk], n_buf.at[q2], in_sem).start()
        pltpu.make_async_copy(energy_hbm.at[chunk], e_buf.at[q2], in_sem).start()

    def _wait_inputs(t, p3, q2):
        chunk = wid + t * NW
        pltpu.make_async_copy(edges_hbm.at[0, chunk], c_buf.at[p3], in_sem).wait()
        pltpu.make_async_copy(edges_hbm.at[1, chunk], n_buf.at[q2], in_sem).wait()
        pltpu.make_async_copy(energy_hbm.at[chunk], e_buf.at[q2], in_sem).wait()

    def _drain_scatters(p3):
        for j in range(ROWS):
            pltpu.make_async_copy(
                vals.at[p3, j], acc.at[c_buf.at[p3, j]], sc_sem).wait()

    _start_inputs(0, 0, 0)

    def _chunk(t, carry):
        p3 = lax.rem(t, 3)
        q2 = lax.rem(t, 2)
        @pl.when(t >= 2)
        def _():
            _drain_scatters(lax.rem(t + 1, 3))

        @pl.when(t + 1 < n_my_chunks)
        def _():
            _start_inputs(t + 1, lax.rem(t + 1, 3), 1 - q2)

        _wait_inputs(t, p3, q2)

        @plsc.parallel_loop(0, CHUNK, 16, unroll=8)
        def _body(i):
            j = lax.shift_right_logical(i, 7)
            sl = pl.ds(lax.bitwise_and(i, 127), 16)
            cc = c_buf[p3, j, sl]
            nn = n_buf[q2, j, sl]
            ee = e_buf[q2, j, sl]
            csp = plsc.load_gather(species_v, [cc])
            nsp = plsc.load_gather(species_v, [nn])
            s = plsc.load_gather(scales_v, [csp * NUM_TYPES + nsp])
            vals[p3, j, sl] = ee * s

        for j in range(ROWS):
            pltpu.async_copy(vals.at[p3, j], acc.at[c_buf.at[p3, j]],
                             sc_sem, add=True)
        return carry

    lax.fori_loop(0, n_my_chunks, _chunk, 0)
    _drain_scatters(lax.rem(n_my_chunks - 2, 3))
    _drain_scatters(lax.rem(n_my_chunks - 1, 3))

    plsc.subcore_barrier()
    pltpu.sync_copy(acc.at[pl.ds(sid * TILE_SLICE, TILE_SLICE)],
                    out_hbm.at[cid, pl.ds(sid * TILE_SLICE, TILE_SLICE)])


def _combine_body(p_ref, o_ref):
    o_ref[...] = (p_ref[0] + p_ref[1]) * FACTOR


_combine = pl.pallas_call(
    _combine_body,
    out_shape=jax.ShapeDtypeStruct((ACC_PAD // 128, 128), jnp.float32),
)


@jax.jit
def kernel(edge_energy, edge_index, atom_types, per_edge_scales):
    edges = edge_index.reshape(2, N_CHUNKS, ROWS, 128)
    energy = edge_energy.reshape(N_CHUNKS, ROWS, 128)
    species = atom_types.reshape(N_NODES)
    scales = per_edge_scales.reshape(NUM_TYPES * NUM_TYPES)
    zeros = jnp.zeros((TILE_SLICE,), jnp.float32)
    partials = _edge_scatter(edges, energy, species, scales, zeros)
    summed = _combine(partials.reshape(2, ACC_PAD // 128, 128))
    return summed.reshape(ACC_PAD)[:N_NODES].reshape(N_NODES, 1)


# speedup vs baseline: 1408.1098x; 1.2802x over previous
import functools

import jax
import jax.numpy as jnp
from jax import lax
from jax.experimental import pallas as pl
from jax.experimental.pallas import tpu as pltpu
from jax.experimental.pallas import tpu_sc as plsc

N_NODES = 100000
N_EDGES = 6400000
NUM_TYPES = 64
FACTOR = 0.125

ROWS = 8
CHUNK = ROWS * 128
N_CHUNKS = N_EDGES // CHUNK
NW = 32
ACC_PAD = 100352
TILE_SLICE = ACC_PAD // 16

_mesh = plsc.VectorSubcoreMesh(core_axis_name="c", subcore_axis_name="s")


@functools.partial(
    pl.kernel,
    mesh=_mesh,
    compiler_params=pltpu.CompilerParams(needs_layout_passes=False),
    out_type=jax.ShapeDtypeStruct((2, ACC_PAD), jnp.float32),
    scratch_types=[
        pltpu.VMEM((N_NODES,), jnp.int32),
        pltpu.VMEM((NUM_TYPES * NUM_TYPES,), jnp.float32),
        pltpu.VMEM((2, CHUNK), jnp.int32),
        pltpu.VMEM((2, CHUNK), jnp.int32),
        pltpu.VMEM((2, CHUNK), jnp.float32),
        pltpu.VMEM((3, ROWS, 128), jnp.int32),
        pltpu.VMEM((3, ROWS, 128), jnp.float32),
        pltpu.VMEM_SHARED((ACC_PAD,), jnp.float32),
        pltpu.SemaphoreType.DMA,
        pltpu.SemaphoreType.DMA,
    ],
)
def _edge_scatter(edges_hbm, energy_hbm, species_hbm,
                  scales_hbm, zeros_hbm, out_hbm,
                  species_v, scales_v, c_in, n_in, e_in, c2, vals,
                  acc, in_sem, sc_sem):
    cid = lax.axis_index("c")
    sid = lax.axis_index("s")
    wid = sid * 2 + cid

    pltpu.sync_copy(zeros_hbm, acc.at[pl.ds(sid * TILE_SLICE, TILE_SLICE)])

    pltpu.sync_copy(species_hbm, species_v)
    pltpu.sync_copy(scales_hbm, scales_v)
    plsc.subcore_barrier()

    n_my_chunks = (N_CHUNKS - 1 - wid) // NW + 1

    def _start_inputs(t, q2):
        base = (wid + t * NW) * CHUNK
        pltpu.make_async_copy(edges_hbm.at[0, pl.ds(base, CHUNK)],
                              c_in.at[q2], in_sem).start()
        pltpu.make_async_copy(edges_hbm.at[1, pl.ds(base, CHUNK)],
                              n_in.at[q2], in_sem).start()
        pltpu.make_async_copy(energy_hbm.at[pl.ds(base, CHUNK)],
                              e_in.at[q2], in_sem).start()

    def _wait_inputs(t, q2):
        base = (wid + t * NW) * CHUNK
        pltpu.make_async_copy(edges_hbm.at[0, pl.ds(base, CHUNK)],
                              c_in.at[q2], in_sem).wait()
        pltpu.make_async_copy(edges_hbm.at[1, pl.ds(base, CHUNK)],
                              n_in.at[q2], in_sem).wait()
        pltpu.make_async_copy(energy_hbm.at[pl.ds(base, CHUNK)],
                              e_in.at[q2], in_sem).wait()

    def _drain_scatters(p3):
        for j in range(ROWS):
            pltpu.make_async_copy(
                vals.at[p3, j], acc.at[c2.at[p3, j]], sc_sem).wait()

    _start_inputs(0, 0)

    def _chunk(t, carry):
        p3 = lax.rem(t, 3)
        q2 = lax.rem(t, 2)
        @pl.when(t >= 2)
        def _():
            _drain_scatters(lax.rem(t + 1, 3))

        @pl.when(t + 1 < n_my_chunks)
        def _():
            _start_inputs(t + 1, 1 - q2)

        _wait_inputs(t, q2)

        @plsc.parallel_loop(0, CHUNK, 16, unroll=8)
        def _body(i):
            cc = c_in[q2, pl.ds(i, 16)]
            nn = n_in[q2, pl.ds(i, 16)]
            ee = e_in[q2, pl.ds(i, 16)]
            csp = plsc.load_gather(species_v, [cc])
            nsp = plsc.load_gather(species_v, [nn])
            s = plsc.load_gather(scales_v, [csp * NUM_TYPES + nsp])
            j = lax.shift_right_logical(i, 7)
            sl = pl.ds(lax.bitwise_and(i, 127), 16)
            c2[p3, j, sl] = cc
            vals[p3, j, sl] = ee * s

        for j in range(ROWS):
            pltpu.async_copy(vals.at[p3, j], acc.at[c2.at[p3, j]],
                             sc_sem, add=True)
        return carry

    lax.fori_loop(0, n_my_chunks, _chunk, 0)
    _drain_scatters(lax.rem(n_my_chunks - 2, 3))
    _drain_scatters(lax.rem(n_my_chunks - 1, 3))

    plsc.subcore_barrier()
    pltpu.sync_copy(acc.at[pl.ds(sid * TILE_SLICE, TILE_SLICE)],
                    out_hbm.at[cid, pl.ds(sid * TILE_SLICE, TILE_SLICE)])


def _combine_body(p_ref, o_ref):
    o_ref[...] = (p_ref[0] + p_ref[1]) * FACTOR


_combine = pl.pallas_call(
    _combine_body,
    out_shape=jax.ShapeDtypeStruct((ACC_PAD,), jnp.float32),
)


@jax.jit
def kernel(edge_energy, edge_index, atom_types, per_edge_scales):
    energy = edge_energy.reshape(N_EDGES)
    species = atom_types.reshape(N_NODES)
    scales = per_edge_scales.reshape(NUM_TYPES * NUM_TYPES)
    zeros = jnp.zeros((TILE_SLICE,), jnp.float32)
    partials = _edge_scatter(edge_index, energy, species, scales, zeros)
    summed = _combine(partials)
    return summed[:N_NODES].reshape(N_NODES, 1)


# speedup vs baseline: 1421.8610x; 1.0098x over previous
import functools

import jax
import jax.numpy as jnp
from jax import lax
from jax.experimental import pallas as pl
from jax.experimental.pallas import tpu as pltpu
from jax.experimental.pallas import tpu_sc as plsc

N_NODES = 100000
N_EDGES = 6400000
NUM_TYPES = 64
FACTOR = 0.125

ROWS = 8
CHUNK = ROWS * 128
N_CHUNKS = N_EDGES // CHUNK
NW = 32
ACC_PAD = 100352
TILE_SLICE = ACC_PAD // 16

_mesh = plsc.VectorSubcoreMesh(core_axis_name="c", subcore_axis_name="s")


@functools.partial(
    pl.kernel,
    mesh=_mesh,
    compiler_params=pltpu.CompilerParams(needs_layout_passes=False),
    out_type=jax.ShapeDtypeStruct((2, ACC_PAD), jnp.float32),
    scratch_types=[
        pltpu.VMEM((N_NODES,), jnp.int32),
        pltpu.VMEM((NUM_TYPES, NUM_TYPES), jnp.float32),
        pltpu.VMEM((2, CHUNK), jnp.int32),
        pltpu.VMEM((2, CHUNK), jnp.int32),
        pltpu.VMEM((2, CHUNK), jnp.float32),
        pltpu.VMEM((3, ROWS, 128), jnp.int32),
        pltpu.VMEM((3, ROWS, 128), jnp.float32),
        pltpu.VMEM_SHARED((ACC_PAD,), jnp.float32),
        pltpu.SemaphoreType.DMA,
        pltpu.SemaphoreType.DMA,
    ],
)
def _edge_scatter(edges_hbm, energy_hbm, species_hbm,
                  scales_hbm, zeros_hbm, out_hbm,
                  species_v, scales_v, c_in, n_in, e_in, c2, vals,
                  acc, in_sem, sc_sem):
    cid = lax.axis_index("c")
    sid = lax.axis_index("s")
    wid = sid * 2 + cid

    pltpu.sync_copy(zeros_hbm, acc.at[pl.ds(sid * TILE_SLICE, TILE_SLICE)])

    pltpu.sync_copy(species_hbm, species_v)
    pltpu.sync_copy(scales_hbm, scales_v)
    plsc.subcore_barrier()

    n_my_chunks = (N_CHUNKS - 1 - wid) // NW + 1

    def _start_inputs(t, q2):
        base = (wid + t * NW) * CHUNK
        pltpu.make_async_copy(edges_hbm.at[0, pl.ds(base, CHUNK)],
                              c_in.at[q2], in_sem).start()
        pltpu.make_async_copy(edges_hbm.at[1, pl.ds(base, CHUNK)],
                              n_in.at[q2], in_sem).start()
        pltpu.make_async_copy(energy_hbm.at[pl.ds(base, CHUNK)],
                              e_in.at[q2], in_sem).start()

    def _wait_inputs(t, q2):
        base = (wid + t * NW) * CHUNK
        pltpu.make_async_copy(edges_hbm.at[0, pl.ds(base, CHUNK)],
                              c_in.at[q2], in_sem).wait()
        pltpu.make_async_copy(edges_hbm.at[1, pl.ds(base, CHUNK)],
                              n_in.at[q2], in_sem).wait()
        pltpu.make_async_copy(energy_hbm.at[pl.ds(base, CHUNK)],
                              e_in.at[q2], in_sem).wait()

    def _drain_scatters(p3):
        for j in range(ROWS):
            pltpu.make_async_copy(
                vals.at[p3, j], acc.at[c2.at[p3, j]], sc_sem).wait()

    _start_inputs(0, 0)

    def _chunk(t, carry):
        p3 = lax.rem(t, 3)
        q2 = lax.rem(t, 2)
        @pl.when(t >= 2)
        def _():
            _drain_scatters(lax.rem(t + 1, 3))

        @pl.when(t + 1 < n_my_chunks)
        def _():
            _start_inputs(t + 1, 1 - q2)

        _wait_inputs(t, q2)

        @plsc.parallel_loop(0, CHUNK, 16, unroll=8)
        def _body(i):
            cc = c_in[q2, pl.ds(i, 16)]
            nn = n_in[q2, pl.ds(i, 16)]
            ee = e_in[q2, pl.ds(i, 16)]
            csp = plsc.load_gather(species_v, [cc])
            nsp = plsc.load_gather(species_v, [nn])
            s = plsc.load_gather(scales_v, [csp, nsp])
            j = lax.shift_right_logical(i, 7)
            sl = pl.ds(lax.bitwise_and(i, 127), 16)
            c2[p3, j, sl] = cc
            vals[p3, j, sl] = ee * s

        for j in range(ROWS):
            pltpu.async_copy(vals.at[p3, j], acc.at[c2.at[p3, j]],
                             sc_sem, add=True)
        return carry

    lax.fori_loop(0, n_my_chunks, _chunk, 0)
    _drain_scatters(lax.rem(n_my_chunks - 2, 3))
    _drain_scatters(lax.rem(n_my_chunks - 1, 3))

    plsc.subcore_barrier()
    pltpu.sync_copy(acc.at[pl.ds(sid * TILE_SLICE, TILE_SLICE)],
                    out_hbm.at[cid, pl.ds(sid * TILE_SLICE, TILE_SLICE)])


def _combine_body(p_ref, o_ref):
    o_ref[...] = (p_ref[0] + p_ref[1]) * FACTOR


_combine = pl.pallas_call(
    _combine_body,
    out_shape=jax.ShapeDtypeStruct((ACC_PAD,), jnp.float32),
)


@jax.jit
def kernel(edge_energy, edge_index, atom_types, per_edge_scales):
    energy = edge_energy.reshape(N_EDGES)
    species = atom_types.reshape(N_NODES)
    zeros = jnp.zeros((TILE_SLICE,), jnp.float32)
    partials = _edge_scatter(edge_index, energy, species,
                             per_edge_scales, zeros)
    summed = _combine(partials)
    return summed[:N_NODES].reshape(N_NODES, 1)
